# baseline (speedup 1.0000x reference)
import dataclasses
import functools

import jax
import jax.numpy as jnp
from jax import lax
from jax.experimental import pallas as pl
from jax.experimental.pallas import tpu as pltpu
from jax.experimental.pallas import tpu_sc as plsc

N_NODES = 10000
N_PAD = 10240
N_EDGES = 320000
E_TOT = N_EDGES + N_NODES
CH = 128
CHH = CH // 2
NC, NS, L = 2, 16, 16
NW = NC * NS
CHUNK = 128
EPS = E_TOT // NS
EPS_PAD = 20992
NCHUNK = EPS_PAD // CHUNK
HCHUNK = NCHUNK // 2
NBUF = 4
ROWS_PT = N_PAD // NS

_mesh = plsc.VectorSubcoreMesh(core_axis_name="c", subcore_axis_name="s")

_cp = pltpu.CompilerParams()
if "needs_layout_passes" in pltpu.CompilerParams.__dataclass_fields__:
    _cp = dataclasses.replace(_cp, needs_layout_passes=False)
_cp = dataclasses.replace(_cp, use_tc_tiling_on_sc=False)


@functools.partial(
    pl.kernel,
    out_type=jax.ShapeDtypeStruct((NW, N_PAD), jnp.float32),
    mesh=_mesh,
    scratch_types=[
        pltpu.VMEM((HCHUNK, CHUNK), jnp.int32),
        pltpu.VMEM((N_PAD,), jnp.float32),
    ],
    compiler_params=_cp,
)
def _hist_kernel(dst3_hbm, out_hbm, idx_v, deg_v):
    cid = lax.axis_index("c")
    sid = lax.axis_index("s")
    wid = cid * NS + sid

    @pl.loop(0, N_PAD, step=L)
    def _(i):
        deg_v[pl.ds(i, L)] = jnp.zeros((L,), jnp.float32)

    pltpu.sync_copy(dst3_hbm.at[sid, pl.ds(cid * HCHUNK, HCHUNK)], idx_v)

    @pl.loop(0, HCHUNK)
    def _(r):
        for c in range(0, CHUNK, L):
            idx = idx_v[r, pl.ds(c, L)]
            cnt, last = plsc.scan_count(idx)
            plsc.addupdate_scatter(
                deg_v, [idx], cnt.astype(jnp.float32), mask=last)

    pltpu.sync_copy(deg_v, out_hbm.at[wid])


@functools.partial(
    pl.kernel,
    out_type=jax.ShapeDtypeStruct((NC, N_PAD, CHH), jnp.float32),
    mesh=_mesh,
    scratch_types=[
        pltpu.VMEM((NCHUNK, CHUNK), jnp.int32),
        pltpu.VMEM((NCHUNK, CHUNK), jnp.int32),
        [pltpu.VMEM((CHUNK, CHH), jnp.float32)] * NBUF,
        pltpu.VMEM_SHARED((N_PAD, CHH), jnp.float32),
        [pltpu.SemaphoreType.DMA] * NBUF,
        [pltpu.SemaphoreType.DMA] * NBUF,
    ],
    compiler_params=_cp,
)
def _scatter_kernel(h2_hbm, src_hbm, dst_hbm, zeros_hbm, out_hbm,
                    src_v, dst_v, bufs, acc, sg, ss):
    cid = lax.axis_index("c")
    sid = lax.axis_index("s")
    h2c = h2_hbm.at[cid]

    pltpu.sync_copy(zeros_hbm, acc.at[pl.ds(sid * ROWS_PT, ROWS_PT)])

    plsc.subcore_barrier()

    pltpu.sync_copy(src_hbm.at[sid], src_v)
    pltpu.sync_copy(dst_hbm.at[sid], dst_v)

    pltpu.async_copy(h2c.at[src_v.at[0]], bufs[0], sg[0])
    pltpu.async_copy(h2c.at[src_v.at[1]], bufs[1], sg[1])

    @pl.loop(0, NCHUNK, step=NBUF)
    def _(j):
        for k in range(NBUF):
            jj = j + k
            kn = (k + 2) % NBUF

            @pl.when(jj >= 2)
            def _():
                pltpu.make_async_copy(
                    bufs[kn], acc.at[dst_v.at[jj - 2]], ss[kn]).wait()

            @pl.when(jj + 2 < NCHUNK)
            def _():
                pltpu.async_copy(h2c.at[src_v.at[jj + 2]], bufs[kn], sg[kn])

            pltpu.make_async_copy(h2c.at[src_v.at[jj]], bufs[k], sg[k]).wait()
            pltpu.async_copy(bufs[k], acc.at[dst_v.at[jj]], ss[k], add=True)

    pltpu.make_async_copy(
        bufs[(NCHUNK - 2) % NBUF], acc.at[dst_v.at[NCHUNK - 2]],
        ss[(NCHUNK - 2) % NBUF]).wait()
    pltpu.make_async_copy(
        bufs[(NCHUNK - 1) % NBUF], acc.at[dst_v.at[NCHUNK - 1]],
        ss[(NCHUNK - 1) % NBUF]).wait()

    plsc.subcore_barrier()
    pltpu.sync_copy(acc.at[pl.ds(sid * ROWS_PT, ROWS_PT)],
                    out_hbm.at[cid, pl.ds(sid * ROWS_PT, ROWS_PT)])


def _mm_body(x_ref, w_ref, h_ref):
    h_ref[...] = jnp.dot(x_ref[...], w_ref[...],
                         preferred_element_type=jnp.float32)


def _scale_body(deg_ref, h_ref, h2_ref):
    dis = lax.rsqrt(jnp.sum(deg_ref[...], axis=0))
    h2 = h_ref[...] * dis[:, None]
    h2_ref[0] = h2[:, :CHH]
    h2_ref[1] = h2[:, CHH:]


def _final_body(deg_ref, acc_ref, b_ref, out_ref):
    dis = lax.rsqrt(jnp.sum(deg_ref[...], axis=0))
    s = jnp.concatenate([acc_ref[0], acc_ref[1]], axis=1)
    out_ref[...] = dis[:, None] * s + b_ref[...]


_RB = 512


def kernel(x, edge_index, W, b):
    loop = jnp.arange(N_NODES, dtype=jnp.int32)
    src = jnp.concatenate([edge_index[0].astype(jnp.int32), loop])
    dst = jnp.concatenate([edge_index[1].astype(jnp.int32), loop])

    npad = EPS_PAD - EPS
    pad_src = jnp.broadcast_to(
        (jnp.arange(npad, dtype=jnp.int32) * 13) % N_NODES, (NS, npad))
    pad_dst = jnp.broadcast_to(
        N_NODES + jnp.arange(npad, dtype=jnp.int32) % (N_PAD - N_NODES),
        (NS, npad))
    src3 = jnp.concatenate(
        [src.reshape(NS, EPS), pad_src], axis=1).reshape(NS, NCHUNK, CHUNK)
    dst3 = jnp.concatenate(
        [dst.reshape(NS, EPS), pad_dst], axis=1).reshape(NS, NCHUNK, CHUNK)

    degp = _hist_kernel(dst3)

    h = pl.pallas_call(
        _mm_body,
        grid=(N_PAD // _RB,),
        in_specs=[
            pl.BlockSpec((_RB, CH), lambda i: (i, 0)),
            pl.BlockSpec((CH, CH), lambda i: (0, 0)),
        ],
        out_specs=pl.BlockSpec((_RB, CH), lambda i: (i, 0)),
        out_shape=jax.ShapeDtypeStruct((N_NODES, CH), jnp.float32),
    )(x, W)

    h2 = pl.pallas_call(
        _scale_body,
        grid=(N_PAD // _RB,),
        in_specs=[
            pl.BlockSpec((NW, _RB), lambda i: (0, i)),
            pl.BlockSpec((_RB, CH), lambda i: (i, 0)),
        ],
        out_specs=pl.BlockSpec((NC, _RB, CHH), lambda i: (0, i, 0)),
        out_shape=jax.ShapeDtypeStruct((NC, N_NODES, CHH), jnp.float32),
    )(degp, h)

    accp = _scatter_kernel(h2, src3, dst3,
                           jnp.zeros((ROWS_PT, CHH), jnp.float32))

    out = pl.pallas_call(
        _final_body,
        grid=(N_PAD // _RB,),
        in_specs=[
            pl.BlockSpec((NW, _RB), lambda i: (0, i)),
            pl.BlockSpec((NC, _RB, CHH), lambda i: (0, i, 0)),
            pl.BlockSpec((1, CH), lambda i: (0, 0)),
        ],
        out_specs=pl.BlockSpec((_RB, CH), lambda i: (i, 0)),
        out_shape=jax.ShapeDtypeStruct((N_NODES, CH), jnp.float32),
    )(degp, accp, b.reshape(1, CH))

    return out


# speedup vs baseline: 1.0463x; 1.0463x over previous
import dataclasses
import functools

import jax
import jax.numpy as jnp
from jax import lax
from jax.experimental import pallas as pl
from jax.experimental.pallas import tpu as pltpu
from jax.experimental.pallas import tpu_sc as plsc

N_NODES = 10000
N_PAD = 10240
N_EDGES = 320000
CH = 128
CHH = CH // 2
NC, NS, L = 2, 16, 16
NW = NC * NS
CHUNK = 128
EPS = N_EDGES // NS
EPS_PAD = 20992
NCHUNK = EPS_PAD // CHUNK
SELF_PS = N_NODES // NS
HCHUNK = NCHUNK // 2
NBUF = 4
ROWS_PT = N_PAD // NS

_mesh = plsc.VectorSubcoreMesh(core_axis_name="c", subcore_axis_name="s")

_cp = pltpu.CompilerParams()
if "needs_layout_passes" in pltpu.CompilerParams.__dataclass_fields__:
    _cp = dataclasses.replace(_cp, needs_layout_passes=False)
_cp = dataclasses.replace(_cp, use_tc_tiling_on_sc=False)


@functools.partial(
    pl.kernel,
    out_type=jax.ShapeDtypeStruct((NW, N_PAD), jnp.float32),
    mesh=_mesh,
    scratch_types=[
        pltpu.VMEM((HCHUNK, CHUNK), jnp.int32),
        pltpu.VMEM((N_PAD,), jnp.float32),
    ],
    compiler_params=_cp,
)
def _hist_kernel(dst3_hbm, out_hbm, idx_v, deg_v):
    cid = lax.axis_index("c")
    sid = lax.axis_index("s")
    wid = cid * NS + sid

    @pl.loop(0, N_PAD, step=L)
    def _(i):
        deg_v[pl.ds(i, L)] = jnp.zeros((L,), jnp.float32)

    pltpu.sync_copy(dst3_hbm.at[sid, pl.ds(cid * HCHUNK, HCHUNK)], idx_v)

    @pl.loop(0, HCHUNK)
    def _(r):
        for c in range(0, CHUNK, L):
            idx = idx_v[r, pl.ds(c, L)]
            cnt, last = plsc.scan_count(idx)
            plsc.addupdate_scatter(
                deg_v, [idx], cnt.astype(jnp.float32), mask=last)

    pltpu.sync_copy(deg_v, out_hbm.at[wid])


@functools.partial(
    pl.kernel,
    out_type=jax.ShapeDtypeStruct((NC, N_PAD, CHH), jnp.float32),
    mesh=_mesh,
    scratch_types=[
        pltpu.VMEM((NCHUNK, CHUNK), jnp.int32),
        pltpu.VMEM((NCHUNK, CHUNK), jnp.int32),
        [pltpu.VMEM((CHUNK, CHH), jnp.float32)] * NBUF,
        pltpu.VMEM_SHARED((N_PAD, CHH), jnp.float32),
        [pltpu.SemaphoreType.DMA] * NBUF,
        [pltpu.SemaphoreType.DMA] * NBUF,
    ],
    compiler_params=_cp,
)
def _scatter_kernel(h2_hbm, src_hbm, dst_hbm, zeros_hbm, out_hbm,
                    src_v, dst_v, bufs, acc, sg, ss):
    cid = lax.axis_index("c")
    sid = lax.axis_index("s")
    h2c = h2_hbm.at[cid]

    pltpu.sync_copy(zeros_hbm, acc.at[pl.ds(sid * ROWS_PT, ROWS_PT)])

    plsc.subcore_barrier()

    pltpu.sync_copy(src_hbm.at[sid], src_v)
    pltpu.sync_copy(dst_hbm.at[sid], dst_v)

    pltpu.async_copy(h2c.at[src_v.at[0]], bufs[0], sg[0])
    pltpu.async_copy(h2c.at[src_v.at[1]], bufs[1], sg[1])

    @pl.loop(0, NCHUNK, step=NBUF)
    def _(j):
        for k in range(NBUF):
            jj = j + k
            kn = (k + 2) % NBUF

            @pl.when(jj >= 2)
            def _():
                pltpu.make_async_copy(
                    bufs[kn], acc.at[dst_v.at[jj - 2]], ss[kn]).wait()

            @pl.when(jj + 2 < NCHUNK)
            def _():
                pltpu.async_copy(h2c.at[src_v.at[jj + 2]], bufs[kn], sg[kn])

            pltpu.make_async_copy(h2c.at[src_v.at[jj]], bufs[k], sg[k]).wait()
            pltpu.async_copy(bufs[k], acc.at[dst_v.at[jj]], ss[k], add=True)

    pltpu.make_async_copy(
        bufs[(NCHUNK - 2) % NBUF], acc.at[dst_v.at[NCHUNK - 2]],
        ss[(NCHUNK - 2) % NBUF]).wait()
    pltpu.make_async_copy(
        bufs[(NCHUNK - 1) % NBUF], acc.at[dst_v.at[NCHUNK - 1]],
        ss[(NCHUNK - 1) % NBUF]).wait()

    plsc.subcore_barrier()
    pltpu.sync_copy(acc.at[pl.ds(sid * ROWS_PT, ROWS_PT)],
                    out_hbm.at[cid, pl.ds(sid * ROWS_PT, ROWS_PT)])


def _mm_body(x_ref, w_ref, h_ref):
    h_ref[...] = jnp.dot(x_ref[...], w_ref[...],
                         preferred_element_type=jnp.float32)


def _scale_body(deg_ref, h_ref, h2_ref):
    dis = lax.rsqrt(jnp.sum(deg_ref[...], axis=0))
    h2 = h_ref[...] * dis[:, None]
    h2_ref[0] = h2[:, :CHH]
    h2_ref[1] = h2[:, CHH:]


def _final_body(deg_ref, acc_ref, b_ref, out_ref):
    dis = lax.rsqrt(jnp.sum(deg_ref[...], axis=0))
    s = jnp.concatenate([acc_ref[0], acc_ref[1]], axis=1)
    out_ref[...] = dis[:, None] * s + b_ref[...]


_RB = 1024


def kernel(x, edge_index, W, b):
    src = edge_index[0].astype(jnp.int32)
    dst = edge_index[1].astype(jnp.int32)

    npad = EPS_PAD - EPS
    ntrash = npad - SELF_PS
    selfs = (jnp.arange(NS, dtype=jnp.int32)[:, None] * SELF_PS
             + jnp.arange(SELF_PS, dtype=jnp.int32)[None, :])
    trash_src = jnp.broadcast_to(
        (jnp.arange(ntrash, dtype=jnp.int32) * 13) % N_NODES, (NS, ntrash))
    trash_dst = jnp.broadcast_to(
        N_NODES + jnp.arange(ntrash, dtype=jnp.int32) % (N_PAD - N_NODES),
        (NS, ntrash))
    src3 = jnp.concatenate(
        [src.reshape(NS, EPS), selfs, trash_src],
        axis=1).reshape(NS, NCHUNK, CHUNK)
    dst3 = jnp.concatenate(
        [dst.reshape(NS, EPS), selfs, trash_dst],
        axis=1).reshape(NS, NCHUNK, CHUNK)

    degp = _hist_kernel(dst3)

    h = pl.pallas_call(
        _mm_body,
        grid=(N_PAD // _RB,),
        in_specs=[
            pl.BlockSpec((_RB, CH), lambda i: (i, 0)),
            pl.BlockSpec((CH, CH), lambda i: (0, 0)),
        ],
        out_specs=pl.BlockSpec((_RB, CH), lambda i: (i, 0)),
        out_shape=jax.ShapeDtypeStruct((N_NODES, CH), jnp.float32),
    )(x, W)

    h2 = pl.pallas_call(
        _scale_body,
        grid=(N_PAD // _RB,),
        in_specs=[
            pl.BlockSpec((NW, _RB), lambda i: (0, i)),
            pl.BlockSpec((_RB, CH), lambda i: (i, 0)),
        ],
        out_specs=pl.BlockSpec((NC, _RB, CHH), lambda i: (0, i, 0)),
        out_shape=jax.ShapeDtypeStruct((NC, N_NODES, CHH), jnp.float32),
    )(degp, h)

    accp = _scatter_kernel(h2, src3, dst3,
                           jnp.zeros((ROWS_PT, CHH), jnp.float32))

    out = pl.pallas_call(
        _final_body,
        grid=(N_PAD // _RB,),
        in_specs=[
            pl.BlockSpec((NW, _RB), lambda i: (0, i)),
            pl.BlockSpec((NC, _RB, CHH), lambda i: (0, i, 0)),
            pl.BlockSpec((1, CH), lambda i: (0, 0)),
        ],
        out_specs=pl.BlockSpec((_RB, CH), lambda i: (i, 0)),
        out_shape=jax.ShapeDtypeStruct((N_NODES, CH), jnp.float32),
    )(degp, accp, b.reshape(1, CH))

    return out


# speedup vs baseline: 1.1289x; 1.0790x over previous
import dataclasses
import functools

import jax
import jax.numpy as jnp
from jax import lax
from jax.experimental import pallas as pl
from jax.experimental.pallas import tpu as pltpu
from jax.experimental.pallas import tpu_sc as plsc

N_NODES = 10000
N_PAD = 10240
N_EDGES = 320000
CH = 128
CHH = CH // 2
NC, NS, L = 2, 16, 16
NW = NC * NS
CHUNK = 128
EPS = N_EDGES // NS
EPS_PAD = 20992
NCHUNK = EPS_PAD // CHUNK
SELF_PS = N_NODES // NS
EPW = N_EDGES // NW
HCHUNK = NCHUNK // 2
NBUF = 4
ROWS_PT = N_PAD // NS

_mesh = plsc.VectorSubcoreMesh(core_axis_name="c", subcore_axis_name="s")

_cp = pltpu.CompilerParams()
if "needs_layout_passes" in pltpu.CompilerParams.__dataclass_fields__:
    _cp = dataclasses.replace(_cp, needs_layout_passes=False)
_cp = dataclasses.replace(_cp, use_tc_tiling_on_sc=False)


@functools.partial(
    pl.kernel,
    out_type=jax.ShapeDtypeStruct((NW, N_PAD), jnp.float32),
    mesh=_mesh,
    scratch_types=[
        pltpu.VMEM((EPW,), jnp.int32),
        pltpu.VMEM((N_PAD,), jnp.float32),
    ],
    compiler_params=_cp,
)
def _hist_kernel(dst_hbm, out_hbm, idx_v, deg_v):
    wid = lax.axis_index("c") * NS + lax.axis_index("s")

    @pl.loop(0, N_PAD, step=L)
    def _(i):
        deg_v[pl.ds(i, L)] = jnp.zeros((L,), jnp.float32)

    pltpu.sync_copy(dst_hbm.at[pl.ds(wid * EPW, EPW)], idx_v)

    @pl.loop(0, EPW, step=L)
    def _(i):
        idx = idx_v[pl.ds(i, L)]
        cnt, last = plsc.scan_count(idx)
        plsc.addupdate_scatter(deg_v, [idx], cnt.astype(jnp.float32), mask=last)

    pltpu.sync_copy(deg_v, out_hbm.at[wid])


@functools.partial(
    pl.kernel,
    out_type=jax.ShapeDtypeStruct((N_PAD, CH), jnp.float32),
    mesh=_mesh,
    scratch_types=[
        pltpu.VMEM((NCHUNK, CHUNK), jnp.int32),
        pltpu.VMEM((NCHUNK, CHUNK), jnp.int32),
        [pltpu.VMEM((CHUNK, CHH), jnp.float32)] * NBUF,
        pltpu.VMEM_SHARED((N_PAD, CHH), jnp.float32),
        [pltpu.SemaphoreType.DMA] * NBUF,
        [pltpu.SemaphoreType.DMA] * NBUF,
    ],
    compiler_params=_cp,
)
def _scatter_kernel(h2_hbm, src_hbm, dst_hbm, zeros_hbm, out_hbm,
                    src_v, dst_v, bufs, acc, sg, ss):
    cid = lax.axis_index("c")
    sid = lax.axis_index("s")
    h2c = h2_hbm.at[cid]

    pltpu.sync_copy(zeros_hbm, acc.at[pl.ds(sid * ROWS_PT, ROWS_PT)])

    plsc.subcore_barrier()

    pltpu.sync_copy(src_hbm.at[sid], src_v)
    pltpu.sync_copy(dst_hbm.at[sid], dst_v)

    pltpu.async_copy(h2c.at[src_v.at[0]], bufs[0], sg[0])
    pltpu.async_copy(h2c.at[src_v.at[1]], bufs[1], sg[1])

    @pl.loop(0, NCHUNK, step=NBUF)
    def _(j):
        for k in range(NBUF):
            jj = j + k
            kn = (k + 2) % NBUF

            @pl.when(jj >= 2)
            def _():
                pltpu.make_async_copy(
                    bufs[kn], acc.at[dst_v.at[jj - 2]], ss[kn]).wait()

            @pl.when(jj + 2 < NCHUNK)
            def _():
                pltpu.async_copy(h2c.at[src_v.at[jj + 2]], bufs[kn], sg[kn])

            pltpu.make_async_copy(h2c.at[src_v.at[jj]], bufs[k], sg[k]).wait()
            pltpu.async_copy(bufs[k], acc.at[dst_v.at[jj]], ss[k], add=True)

    pltpu.make_async_copy(
        bufs[(NCHUNK - 2) % NBUF], acc.at[dst_v.at[NCHUNK - 2]],
        ss[(NCHUNK - 2) % NBUF]).wait()
    pltpu.make_async_copy(
        bufs[(NCHUNK - 1) % NBUF], acc.at[dst_v.at[NCHUNK - 1]],
        ss[(NCHUNK - 1) % NBUF]).wait()

    plsc.subcore_barrier()
    pltpu.sync_copy(acc.at[pl.ds(sid * ROWS_PT, ROWS_PT)],
                    out_hbm.at[pl.ds(sid * ROWS_PT, ROWS_PT),
                               pl.ds(cid * CHH, CHH)])


def _mm_body(x_ref, w_ref, h_ref):
    h_ref[...] = jnp.dot(x_ref[...], w_ref[...],
                         preferred_element_type=jnp.float32)


def _scale_body(deg_ref, h_ref, h2_ref):
    dis = lax.rsqrt(jnp.sum(deg_ref[...], axis=0) + 1.0)
    h2 = h_ref[...] * dis[:, None]
    h2_ref[0] = h2[:, :CHH]
    h2_ref[1] = h2[:, CHH:]


def _final_body(deg_ref, acc_ref, b_ref, out_ref):
    dis = lax.rsqrt(jnp.sum(deg_ref[...], axis=0) + 1.0)
    out_ref[...] = dis[:, None] * acc_ref[...] + b_ref[...]


_RB = 1024


def kernel(x, edge_index, W, b):
    src = edge_index[0].astype(jnp.int32)
    dst = edge_index[1].astype(jnp.int32)

    npad = EPS_PAD - EPS
    ntrash = npad - SELF_PS
    selfs = (jnp.arange(NS, dtype=jnp.int32)[:, None] * SELF_PS
             + jnp.arange(SELF_PS, dtype=jnp.int32)[None, :])
    trash_src = jnp.broadcast_to(
        (jnp.arange(ntrash, dtype=jnp.int32) * 13) % N_NODES, (NS, ntrash))
    trash_dst = jnp.broadcast_to(
        N_NODES + jnp.arange(ntrash, dtype=jnp.int32) % (N_PAD - N_NODES),
        (NS, ntrash))
    src3 = jnp.concatenate(
        [src.reshape(NS, EPS), selfs, trash_src],
        axis=1).reshape(NS, NCHUNK, CHUNK)
    dst3 = jnp.concatenate(
        [dst.reshape(NS, EPS), selfs, trash_dst],
        axis=1).reshape(NS, NCHUNK, CHUNK)

    degp = _hist_kernel(dst)

    h = pl.pallas_call(
        _mm_body,
        grid=(N_PAD // _RB,),
        in_specs=[
            pl.BlockSpec((_RB, CH), lambda i: (i, 0)),
            pl.BlockSpec((CH, CH), lambda i: (0, 0)),
        ],
        out_specs=pl.BlockSpec((_RB, CH), lambda i: (i, 0)),
        out_shape=jax.ShapeDtypeStruct((N_NODES, CH), jnp.float32),
    )(x, W)

    h2 = pl.pallas_call(
        _scale_body,
        grid=(N_PAD // _RB,),
        in_specs=[
            pl.BlockSpec((NW, _RB), lambda i: (0, i)),
            pl.BlockSpec((_RB, CH), lambda i: (i, 0)),
        ],
        out_specs=pl.BlockSpec((NC, _RB, CHH), lambda i: (0, i, 0)),
        out_shape=jax.ShapeDtypeStruct((NC, N_NODES, CHH), jnp.float32),
    )(degp, h)

    accp = _scatter_kernel(h2, src3, dst3,
                           jnp.zeros((ROWS_PT, CHH), jnp.float32))

    out = pl.pallas_call(
        _final_body,
        grid=(N_PAD // _RB,),
        in_specs=[
            pl.BlockSpec((NW, _RB), lambda i: (0, i)),
            pl.BlockSpec((_RB, CH), lambda i: (i, 0)),
            pl.BlockSpec((1, CH), lambda i: (0, 0)),
        ],
        out_specs=pl.BlockSpec((_RB, CH), lambda i: (i, 0)),
        out_shape=jax.ShapeDtypeStruct((N_NODES, CH), jnp.float32),
    )(degp, accp, b.reshape(1, CH))

    return out
